# baseline (speedup 1.0000x reference)
import dataclasses
import functools

import jax
import jax.numpy as jnp
from jax import lax
from jax.experimental import pallas as pl
from jax.experimental.pallas import tpu as pltpu
from jax.experimental.pallas import tpu_sc as plsc

LANES = 16
STREAM_IDX = 128


def _tree_sum(vals):
  vals = list(vals)
  while len(vals) > 1:
    nxt = [a + b for a, b in zip(vals[0::2], vals[1::2])]
    if len(vals) % 2:
      nxt.append(vals[-1])
    vals = nxt
  return vals[0]


def _compiler_params(tc_tiling):
  cp = pltpu.CompilerParams()
  fields = getattr(pltpu.CompilerParams, "__dataclass_fields__", {})
  if "needs_layout_passes" in fields:
    cp = dataclasses.replace(cp, needs_layout_passes=False)
  if "use_tc_tiling_on_sc" in fields:
    cp = dataclasses.replace(cp, use_tc_tiling_on_sc=tc_tiling)
  return cp


def _make_mean_call(B, CTX, D, dtype):
  mesh = plsc.VectorSubcoreMesh(core_axis_name="c", subcore_axis_name="s")
  NC = mesh.num_cores
  NW = NC * mesh.num_subcores
  PER_W = B // NW
  C = LANES
  NCHUNK = PER_W // C
  assert NCHUNK % 2 == 0

  def body(ctx_idx_hbm, emb_in_hbm, mean_hbm,
           ctx_idx_v, ctx_rows, mean_buf, sems):
    cid = lax.axis_index("c")
    sid = lax.axis_index("s")
    wid = sid * NC + cid
    base = wid * PER_W

    pltpu.sync_copy(ctx_idx_hbm.at[pl.ds(base * CTX, PER_W * CTX)], ctx_idx_v)

    def issue(ci, b):
      co = ci * C
      descs = []
      for o in range(0, C * CTX, STREAM_IDX):
        w = min(STREAM_IDX, C * CTX - o)
        descs.append(pltpu.make_async_copy(
            emb_in_hbm.at[ctx_idx_v.at[pl.ds(co * CTX + o, w)]],
            ctx_rows.at[b].at[pl.ds(o, w)], sems.at[b]))
      for d_ in descs:
        d_.start()
      return descs

    def compute(ci, b):
      co = ci * C
      rows = ctx_rows.at[b]

      @pl.loop(0, C)
      def _el(e):
        r0 = e * CTX
        for dc in range(D // LANES):
          acc = _tree_sum([rows[r0 + j, pl.ds(dc * LANES, LANES)]
                           for j in range(CTX)])
          mean_buf[pl.ds(e * D + dc * LANES, LANES)] = acc * (1.0 / CTX)

      pltpu.sync_copy(mean_buf, mean_hbm.at[pl.ds((base + co) * D, C * D)])

    d0 = issue(0, 0)

    @pl.loop(0, NCHUNK, step=2)
    def _chunk(ci):
      issue(ci + 1, 1)
      co = ci * C
      for o in range(0, C * CTX, STREAM_IDX):
        w = min(STREAM_IDX, C * CTX - o)
        pltpu.make_async_copy(
            emb_in_hbm.at[ctx_idx_v.at[pl.ds(co * CTX + o, w)]],
            ctx_rows.at[0].at[pl.ds(o, w)], sems.at[0]).wait()
      compute(ci, 0)

      @pl.when(ci + 2 < NCHUNK)
      def _():
        issue(ci + 2, 0)

      co1 = (ci + 1) * C
      for o in range(0, C * CTX, STREAM_IDX):
        w = min(STREAM_IDX, C * CTX - o)
        pltpu.make_async_copy(
            emb_in_hbm.at[ctx_idx_v.at[pl.ds(co1 * CTX + o, w)]],
            ctx_rows.at[1].at[pl.ds(o, w)], sems.at[1]).wait()
      compute(ci + 1, 1)

  return pl.kernel(
      body,
      out_type=jax.ShapeDtypeStruct((B * D,), dtype),
      mesh=mesh,
      compiler_params=_compiler_params(False),
      scratch_types=[
          pltpu.VMEM((PER_W * CTX,), jnp.int32),
          pltpu.VMEM((2, C * CTX, D), dtype),
          pltpu.VMEM((C * D,), dtype),
          pltpu.SemaphoreType.DMA((2,)),
      ],
  )


def _make_score_call(B, NEG, D, dtype):
  mesh = plsc.VectorSubcoreMesh(core_axis_name="c", subcore_axis_name="s")
  NC = mesh.num_cores
  NW = NC * mesh.num_subcores
  PER_W = B // NW
  C = LANES
  NCHUNK = PER_W // C
  assert NCHUNK % 2 == 0
  NROW = C * (NEG + 1)
  WAVE = 4 * LANES

  def body(tgt_idx_hbm, neg_idx_hbm, emb_out_hbm, mean_hbm,
           pos_hbm, neg_hbm,
           tgt_idx_v, neg_idx_v, rows, mean_v, pos_buf, neg_buf, sems):
    cid = lax.axis_index("c")
    sid = lax.axis_index("s")
    wid = sid * NC + cid
    base = wid * PER_W

    pltpu.sync_copy(tgt_idx_hbm.at[pl.ds(base, PER_W)], tgt_idx_v)
    pltpu.sync_copy(neg_idx_hbm.at[pl.ds(base * NEG, PER_W * NEG)], neg_idx_v)

    e_iota = lax.iota(jnp.int32, LANES)

    def row_waves(ci, b):
      co = ci * C
      waves = []
      for idx_ref, off, r0, n in (
          (neg_idx_v, co * NEG, 0, C * NEG),
          (tgt_idx_v, co, C * NEG, C)):
        for w0 in range(0, n, WAVE):
          waves.append((idx_ref, off, r0, w0, min(WAVE, n - w0)))
      return waves

    def fire(wave, b):
      idx_ref, off, r0, w0, n = wave
      for g in range(0, n, LANES):
        vec = idx_ref[pl.ds(off + w0 + g, LANES)]
        for l in range(LANES):
          v = lax.squeeze(lax.slice(vec, (l,), (l + 1,)), (0,))
          pltpu.make_async_copy(
              emb_out_hbm.at[pl.ds(v, 1)],
              rows.at[b].at[pl.ds(r0 + w0 + g + l, 1)], sems.at[b]).start()

    def drain_wave(wave, b):
      idx_ref, off, r0, w0, n = wave
      pltpu.make_async_copy(emb_out_hbm.at[pl.ds(0, n)],
                            rows.at[b].at[pl.ds(r0 + w0, n)],
                            sems.at[b]).wait()

    def issue(ci, b, ahead=2):
      co = ci * C
      pltpu.make_async_copy(mean_hbm.at[pl.ds(base + co, C)],
                            mean_v.at[b], sems.at[b]).start()
      waves = row_waves(ci, b)
      for i, w in enumerate(waves):
        fire(w, b)
        if i >= ahead:
          drain_wave(waves[i - ahead], b)
      return waves[len(waves) - ahead:]

    def drain(tail, ci, b):
      for w in tail:
        drain_wave(w, b)
      co = ci * C
      pltpu.make_async_copy(mean_hbm.at[pl.ds(base + co, C)],
                            mean_v.at[b], sems.at[b]).wait()

    row_neg = [e_iota * NEG + n for n in range(NEG)]
    row_tgt = C * NEG + e_iota

    def compute(ci, b):
      co = ci * C
      rws = rows.at[b]
      mv = mean_v.at[b]

      def dbody(d, carry):
        pos_acc, neg_accs = carry
        cold = (e_iota + d) & (D - 1)
        m = plsc.load_gather(mv, [e_iota, cold])
        pos_acc = pos_acc + plsc.load_gather(rws, [row_tgt, cold]) * m
        neg_accs = tuple(
            neg_accs[n] + plsc.load_gather(rws, [row_neg[n], cold]) * m
            for n in range(NEG))
        return pos_acc, neg_accs

      zero = jnp.zeros((LANES,), jnp.float32)
      pos_acc, neg_accs = lax.fori_loop(0, D, dbody, (zero, (zero,) * NEG))

      pos_buf[...] = pos_acc
      for n in range(NEG):
        plsc.store_scatter(neg_buf, [e_iota * NEG + n], neg_accs[n])
      pltpu.sync_copy(pos_buf, pos_hbm.at[pl.ds(base + co, C)])
      pltpu.sync_copy(neg_buf, neg_hbm.at[pl.ds((base + co) * NEG, C * NEG)])

    tail0 = issue(0, 0)

    @pl.loop(0, NCHUNK, step=2)
    def _chunk(ci):
      tail1 = issue(ci + 1, 1)
      drain(tail0, ci, 0)
      compute(ci, 0)

      @pl.when(ci + 2 < NCHUNK)
      def _():
        issue(ci + 2, 0)

      drain(tail1, ci + 1, 1)
      compute(ci + 1, 1)

  return pl.kernel(
      body,
      out_type=(jax.ShapeDtypeStruct((B,), dtype),
                jax.ShapeDtypeStruct((B * NEG,), dtype)),
      mesh=mesh,
      compiler_params=_compiler_params(True),
      scratch_types=[
          pltpu.VMEM((PER_W,), jnp.int32),
          pltpu.VMEM((PER_W * NEG,), jnp.int32),
          pltpu.VMEM((2, NROW, D), dtype),
          pltpu.VMEM((2, C, D), dtype),
          pltpu.VMEM((LANES,), dtype),
          pltpu.VMEM((C * NEG,), dtype),
          pltpu.SemaphoreType.DMA((2,)),
      ],
  )


def kernel(input_ids, labels, negative_samples, emb_in, emb_out):
  B, CTX = input_ids.shape
  NEG = negative_samples.shape[1]
  V, D = emb_in.shape
  ctx_idx = input_ids.reshape(-1).astype(jnp.int32)
  tgt_idx = labels.reshape(-1).astype(jnp.int32)
  neg_idx = negative_samples.reshape(-1).astype(jnp.int32)
  mean_flat = _make_mean_call(B, CTX, D, emb_in.dtype)(ctx_idx, emb_in)
  pos, neg = _make_score_call(B, NEG, D, emb_in.dtype)(
      tgt_idx, neg_idx, emb_out, mean_flat.reshape(B, D))
  return pos, neg.reshape(B, NEG)


# speedup vs baseline: 1.2633x; 1.2633x over previous
import dataclasses
import functools

import jax
import jax.numpy as jnp
from jax import lax
from jax.experimental import pallas as pl
from jax.experimental.pallas import tpu as pltpu
from jax.experimental.pallas import tpu_sc as plsc

LANES = 16
STREAM_IDX = 128


def _tree_sum(vals):
  vals = list(vals)
  while len(vals) > 1:
    nxt = [a + b for a, b in zip(vals[0::2], vals[1::2])]
    if len(vals) % 2:
      nxt.append(vals[-1])
    vals = nxt
  return vals[0]


def _compiler_params(tc_tiling):
  cp = pltpu.CompilerParams()
  fields = getattr(pltpu.CompilerParams, "__dataclass_fields__", {})
  if "needs_layout_passes" in fields:
    cp = dataclasses.replace(cp, needs_layout_passes=False)
  if "use_tc_tiling_on_sc" in fields:
    cp = dataclasses.replace(cp, use_tc_tiling_on_sc=tc_tiling)
  return cp


def _make_mean_call(B, CTX, D, dtype):
  mesh = plsc.VectorSubcoreMesh(core_axis_name="c", subcore_axis_name="s")
  NC = mesh.num_cores
  NW = NC * mesh.num_subcores
  PER_W = B // NW
  C = LANES
  NCHUNK = PER_W // C
  assert NCHUNK % 2 == 0
  NROW = C * CTX
  WAVE = 4 * LANES

  def body(ctx_idx_hbm, emb_in_hbm, mean_hbm,
           ctx_idx_v, ctx_rows, mean_buf, sems):
    cid = lax.axis_index("c")
    sid = lax.axis_index("s")
    wid = sid * NC + cid
    base = wid * PER_W

    pltpu.sync_copy(ctx_idx_hbm.at[pl.ds(base * CTX, PER_W * CTX)], ctx_idx_v)

    e_iota = lax.iota(jnp.int32, LANES)
    waves = [(w0, min(WAVE, NROW - w0)) for w0 in range(0, NROW, WAVE)]

    def fire(ci, b, w0, n):
      co = ci * C
      for g in range(0, n, LANES):
        vec = ctx_idx_v[pl.ds(co * CTX + w0 + g, LANES)]
        for l in range(LANES):
          v = lax.squeeze(lax.slice(vec, (l,), (l + 1,)), (0,))
          pltpu.make_async_copy(
              emb_in_hbm.at[pl.ds(v, 1)],
              ctx_rows.at[b].at[pl.ds(w0 + g + l, 1)], sems.at[b]).start()

    def drain_wave(b, w0, n):
      pltpu.make_async_copy(emb_in_hbm.at[pl.ds(0, n)],
                            ctx_rows.at[b].at[pl.ds(w0, n)],
                            sems.at[b]).wait()

    def issue(ci, b, ahead=2):
      for i, (w0, n) in enumerate(waves):
        fire(ci, b, w0, n)
        if i >= ahead:
          drain_wave(b, *waves[i - ahead])

    def drain(b, ahead=2):
      for w0, n in waves[len(waves) - ahead:]:
        drain_wave(b, w0, n)

    row_ctx = [e_iota * CTX + j for j in range(CTX)]

    def compute(ci, b):
      co = ci * C
      rows = ctx_rows.at[b]

      @pl.loop(0, D)
      def _mean(d):
        cold = (e_iota + d) & (D - 1)
        m = _tree_sum([plsc.load_gather(rows, [row_ctx[j], cold])
                       for j in range(CTX)])
        plsc.store_scatter(mean_buf, [e_iota * D + cold], m * (1.0 / CTX))

      pltpu.sync_copy(mean_buf, mean_hbm.at[pl.ds((base + co) * D, C * D)])

    issue(0, 0)

    @pl.loop(0, NCHUNK, step=2)
    def _chunk(ci):
      issue(ci + 1, 1)
      drain(0)
      compute(ci, 0)

      @pl.when(ci + 2 < NCHUNK)
      def _():
        issue(ci + 2, 0)

      drain(1)
      compute(ci + 1, 1)

  return pl.kernel(
      body,
      out_type=jax.ShapeDtypeStruct((B * D,), dtype),
      mesh=mesh,
      compiler_params=_compiler_params(True),
      scratch_types=[
          pltpu.VMEM((PER_W * CTX,), jnp.int32),
          pltpu.VMEM((2, NROW, D), dtype),
          pltpu.VMEM((C * D,), dtype),
          pltpu.SemaphoreType.DMA((2,)),
      ],
  )


def _make_score_call(B, NEG, D, dtype):
  mesh = plsc.VectorSubcoreMesh(core_axis_name="c", subcore_axis_name="s")
  NC = mesh.num_cores
  NW = NC * mesh.num_subcores
  PER_W = B // NW
  C = LANES
  NCHUNK = PER_W // C
  assert NCHUNK % 2 == 0
  NROW = C * (NEG + 1)
  WAVE = 4 * LANES

  def body(tgt_idx_hbm, neg_idx_hbm, emb_out_hbm, mean_hbm,
           pos_hbm, neg_hbm,
           tgt_idx_v, neg_idx_v, rows, mean_v, pos_buf, neg_buf, sems):
    cid = lax.axis_index("c")
    sid = lax.axis_index("s")
    wid = sid * NC + cid
    base = wid * PER_W

    pltpu.sync_copy(tgt_idx_hbm.at[pl.ds(base, PER_W)], tgt_idx_v)
    pltpu.sync_copy(neg_idx_hbm.at[pl.ds(base * NEG, PER_W * NEG)], neg_idx_v)

    e_iota = lax.iota(jnp.int32, LANES)

    def row_waves(ci, b):
      co = ci * C
      waves = []
      for idx_ref, off, r0, n in (
          (neg_idx_v, co * NEG, 0, C * NEG),
          (tgt_idx_v, co, C * NEG, C)):
        for w0 in range(0, n, WAVE):
          waves.append((idx_ref, off, r0, w0, min(WAVE, n - w0)))
      return waves

    def fire(wave, b):
      idx_ref, off, r0, w0, n = wave
      for g in range(0, n, LANES):
        vec = idx_ref[pl.ds(off + w0 + g, LANES)]
        for l in range(LANES):
          v = lax.squeeze(lax.slice(vec, (l,), (l + 1,)), (0,))
          pltpu.make_async_copy(
              emb_out_hbm.at[pl.ds(v, 1)],
              rows.at[b].at[pl.ds(r0 + w0 + g + l, 1)], sems.at[b]).start()

    def drain_wave(wave, b):
      idx_ref, off, r0, w0, n = wave
      pltpu.make_async_copy(emb_out_hbm.at[pl.ds(0, n)],
                            rows.at[b].at[pl.ds(r0 + w0, n)],
                            sems.at[b]).wait()

    def issue(ci, b, ahead=2):
      co = ci * C
      pltpu.make_async_copy(mean_hbm.at[pl.ds(base + co, C)],
                            mean_v.at[b], sems.at[b]).start()
      waves = row_waves(ci, b)
      for i, w in enumerate(waves):
        fire(w, b)
        if i >= ahead:
          drain_wave(waves[i - ahead], b)
      return waves[len(waves) - ahead:]

    def drain(tail, ci, b):
      for w in tail:
        drain_wave(w, b)
      co = ci * C
      pltpu.make_async_copy(mean_hbm.at[pl.ds(base + co, C)],
                            mean_v.at[b], sems.at[b]).wait()

    row_neg = [e_iota * NEG + n for n in range(NEG)]
    row_tgt = C * NEG + e_iota

    def compute(ci, b):
      co = ci * C
      rws = rows.at[b]
      mv = mean_v.at[b]

      def dbody(d, carry):
        pos_acc, neg_accs = carry
        cold = (e_iota + d) & (D - 1)
        m = plsc.load_gather(mv, [e_iota, cold])
        pos_acc = pos_acc + plsc.load_gather(rws, [row_tgt, cold]) * m
        neg_accs = tuple(
            neg_accs[n] + plsc.load_gather(rws, [row_neg[n], cold]) * m
            for n in range(NEG))
        return pos_acc, neg_accs

      zero = jnp.zeros((LANES,), jnp.float32)
      pos_acc, neg_accs = lax.fori_loop(0, D, dbody, (zero, (zero,) * NEG))

      pos_buf[...] = pos_acc
      for n in range(NEG):
        plsc.store_scatter(neg_buf, [e_iota * NEG + n], neg_accs[n])
      pltpu.sync_copy(pos_buf, pos_hbm.at[pl.ds(base + co, C)])
      pltpu.sync_copy(neg_buf, neg_hbm.at[pl.ds((base + co) * NEG, C * NEG)])

    tail0 = issue(0, 0)

    @pl.loop(0, NCHUNK, step=2)
    def _chunk(ci):
      tail1 = issue(ci + 1, 1)
      drain(tail0, ci, 0)
      compute(ci, 0)

      @pl.when(ci + 2 < NCHUNK)
      def _():
        issue(ci + 2, 0)

      drain(tail1, ci + 1, 1)
      compute(ci + 1, 1)

  return pl.kernel(
      body,
      out_type=(jax.ShapeDtypeStruct((B,), dtype),
                jax.ShapeDtypeStruct((B * NEG,), dtype)),
      mesh=mesh,
      compiler_params=_compiler_params(True),
      scratch_types=[
          pltpu.VMEM((PER_W,), jnp.int32),
          pltpu.VMEM((PER_W * NEG,), jnp.int32),
          pltpu.VMEM((2, NROW, D), dtype),
          pltpu.VMEM((2, C, D), dtype),
          pltpu.VMEM((LANES,), dtype),
          pltpu.VMEM((C * NEG,), dtype),
          pltpu.SemaphoreType.DMA((2,)),
      ],
  )


def kernel(input_ids, labels, negative_samples, emb_in, emb_out):
  B, CTX = input_ids.shape
  NEG = negative_samples.shape[1]
  V, D = emb_in.shape
  ctx_idx = input_ids.reshape(-1).astype(jnp.int32)
  tgt_idx = labels.reshape(-1).astype(jnp.int32)
  neg_idx = negative_samples.reshape(-1).astype(jnp.int32)
  mean_flat = _make_mean_call(B, CTX, D, emb_in.dtype)(ctx_idx, emb_in)
  pos, neg = _make_score_call(B, NEG, D, emb_in.dtype)(
      tgt_idx, neg_idx, emb_out, mean_flat.reshape(B, D))
  return pos, neg.reshape(B, NEG)
